# baseline (speedup 1.0000x reference)
import math
import statistics

import jax
import jax.numpy as jnp
from jax import lax
from jax.experimental import pallas as pl
from jax.experimental.pallas import tpu as pltpu

SPARSITY_RATIO = 0.5
VISION_SPARSITY_RATIO = 0.4


def _to_key(bits):
    return jnp.where(bits >= 0, bits, bits ^ jnp.int32(0x7FFFFFFF))


def _proj_body(nheads, head_dim, hs, wq, wk, wv, wr, bq, bk, bv, br, cos, sin,
               q_o, k_o, v_o, r_o):
    j = pl.program_id(1)
    x = hs[...]
    qt = jnp.dot(x, wq[...], preferred_element_type=jnp.float32) + bq[...]
    kt = jnp.dot(x, wk[...], preferred_element_type=jnp.float32) + bk[...]
    cosb = cos[...]
    sinb = sin[...]
    hd = head_dim
    half = hd // 2
    for hh in range(qt.shape[1] // hd):
        sl = slice(hh * hd, (hh + 1) * hd)
        qh = qt[:, sl]
        qr = jnp.concatenate([-qh[:, half:], qh[:, :half]], axis=1)
        q_o[:, sl] = qh * cosb + qr * sinb
        kh = kt[:, sl]
        kr = jnp.concatenate([-kh[:, half:], kh[:, :half]], axis=1)
        k_o[:, sl] = kh * cosb + kr * sinb
    v_o[...] = jnp.dot(x, wv[...], preferred_element_type=jnp.float32) + bv[...]

    @pl.when(j == 0)
    def _():
        r_o[...] = jax.nn.sigmoid(
            jnp.dot(x, wr[...], preferred_element_type=jnp.float32) + br[...])


def _attn_body(keep, kt_tile, q_r, k_r, v_r, r_r, o_r, m_scr):
    h = pl.program_id(0)
    bq = q_r.shape[0]
    qlen = k_r.shape[0]
    nkt = qlen // kt_tile
    hd = q_r.shape[1]
    scale = jnp.float32(1.0 / math.sqrt(hd))

    qv = q_r[...]
    rall = r_r[...]
    lane = lax.broadcasted_iota(jnp.int32, rall.shape, 1)
    rcol = jnp.sum(jnp.where(lane == h, rall, 0.0), axis=1, keepdims=True)
    rinv = 1.0 / rcol

    mmax = None
    mmin = None
    msum = None
    msum2 = None
    for t in range(nkt):
        ksl = pl.ds(t * kt_tile, kt_tile)
        kv = k_r[ksl, :]
        st = lax.dot_general(qv, kv, (((1,), (1,)), ((), ())),
                             preferred_element_type=jnp.float32) * scale
        mt = st * rcol
        m_scr[:, ksl] = mt
        tmax = jnp.max(mt, axis=1, keepdims=True)
        tmin = jnp.min(mt, axis=1, keepdims=True)
        tsum = jnp.sum(mt, axis=1, keepdims=True)
        tsum2 = jnp.sum(mt * mt, axis=1, keepdims=True)
        mmax = tmax if mmax is None else jnp.maximum(mmax, tmax)
        mmin = tmin if mmin is None else jnp.minimum(mmin, tmin)
        msum = tsum if msum is None else msum + tsum
        msum2 = tsum2 if msum2 is None else msum2 + tsum2

    kf = jnp.float32(keep)
    lo = _to_key(lax.bitcast_convert_type(mmin, jnp.int32))
    hi = _to_key(lax.bitcast_convert_type(mmax, jnp.int32))
    clo = jnp.full((bq, 1), jnp.float32(qlen))
    chi = jnp.zeros((bq, 1), jnp.float32)

    def update2(st_, mid_a, mid_b):
        lo_, hi_, clo_, chi_ = st_
        a = jnp.clip(jnp.minimum(mid_a, mid_b), lo_ + 1, hi_)
        b = jnp.clip(jnp.maximum(mid_a, mid_b), lo_ + 1, hi_)
        va = lax.bitcast_convert_type(_to_key(a), jnp.float32)
        vb = lax.bitcast_convert_type(_to_key(b), jnp.float32)
        ca = jnp.zeros((bq, 1), jnp.float32)
        cb = jnp.zeros((bq, 1), jnp.float32)
        for t in range(nkt):
            mt = m_scr[:, pl.ds(t * kt_tile, kt_tile)]
            ca = ca + jnp.sum(jnp.where(mt >= va, 1.0, 0.0),
                              axis=1, keepdims=True)
            cb = cb + jnp.sum(jnp.where(mt >= vb, 1.0, 0.0),
                              axis=1, keepdims=True)
        lo2 = jnp.where(cb >= kf, b, jnp.where(ca >= kf, a, lo_))
        clo2 = jnp.where(cb >= kf, cb, jnp.where(ca >= kf, ca, clo_))
        hi2 = jnp.where(ca < kf, a - 1, jnp.where(cb < kf, b - 1, hi_))
        chi2 = jnp.where(ca < kf, ca, jnp.where(cb < kf, cb, chi_))
        fr = jnp.where(ca == kf, a, b)
        hit = (ca == kf) | (cb == kf)
        lo2 = jnp.where(hit, fr, lo2)
        hi2 = jnp.where(hit, fr, hi2)
        return (lo2, hi2, clo2, chi2)

    def interp_mid(st_, off):
        lo_, hi_, clo_, chi_ = st_
        vlo = lax.bitcast_convert_type(_to_key(lo_), jnp.float32)
        vhi = lax.bitcast_convert_type(_to_key(hi_), jnp.float32)
        span = jnp.maximum(clo_ - chi_, 1.0)
        frac = (clo_ - (kf + off * jnp.maximum(span * 0.04, 0.6))) / span
        return _to_key(lax.bitcast_convert_type(vlo + frac * (vhi - vlo),
                                                jnp.int32))

    def bisect_mid(st_):
        lo_, hi_, _, _ = st_
        return (lo_ >> 1) + (hi_ >> 1) + ((lo_ | hi_) & 1)

    pq = (qlen - keep) / qlen
    nd = statistics.NormalDist()
    za = jnp.float32(nd.inv_cdf(max(1e-6, min(1.0 - 1e-6, pq - 0.01))))
    zb = jnp.float32(nd.inv_cdf(max(1e-6, min(1.0 - 1e-6, pq + 0.01))))
    mu = msum * jnp.float32(1.0 / qlen)
    sd = jnp.sqrt(jnp.maximum(msum2 * jnp.float32(1.0 / qlen) - mu * mu, 0.0))
    state = (lo, hi, clo, chi)
    state = update2(state,
                    _to_key(lax.bitcast_convert_type(mu + sd * za, jnp.int32)),
                    _to_key(lax.bitcast_convert_type(mu + sd * zb, jnp.int32)))
    for n in range(3):
        state = update2(state, interp_mid(state, jnp.float32(1.0)),
                        interp_mid(state, jnp.float32(-1.0)))

    def cond(st_):
        lo_, hi_, _, _ = st_
        return jnp.any(lo_ < hi_)

    def wbody(st_):
        return update2(st_, interp_mid(st_, jnp.float32(0.0)),
                       bisect_mid(st_))

    lo, hi, _, _ = lax.while_loop(cond, wbody, state)
    vT = lax.bitcast_convert_type(_to_key(lo), jnp.float32)

    smax = mmax * rinv
    denom = jnp.zeros((bq, 1), jnp.float32)
    acc = jnp.zeros((bq, hd), jnp.float32)
    for t in range(nkt):
        ksl = pl.ds(t * kt_tile, kt_tile)
        mt = m_scr[:, ksl]
        st = mt * rinv
        p = jnp.where(mt >= vT, jnp.exp(st - smax), 0.0)
        denom = denom + jnp.sum(p, axis=1, keepdims=True)
        acc = acc + jnp.dot(p, v_r[ksl, :], preferred_element_type=jnp.float32)
    o_r[...] = acc / denom


def _outproj_body(x, wo, bo, o):
    o[...] = jnp.dot(x[...], wo[...], preferred_element_type=jnp.float32) + bo[...]


def kernel(hidden_states, Wq, bq, Wk, bk, Wv, bv, Wo, bo, Wr, br):
    bsz, q_len, hidden = hidden_states.shape
    nheads = Wr.shape[1]
    hd = Wq.shape[1] // nheads
    ratio = VISION_SPARSITY_RATIO if q_len > 512 else SPARSITY_RATIO
    keep = max(1, int(q_len * (1.0 - ratio)))

    BQ = min(512, q_len)
    BN = min(256, nheads * hd)
    KT = min(512, q_len)
    n_qb = q_len // BQ
    n_nb = (nheads * hd) // BN

    x = hidden_states.reshape(q_len, hidden)

    pos = jnp.arange(q_len, dtype=jnp.float32)
    inv_freq = 1.0 / (10000.0 ** (jnp.arange(0, hd, 2, dtype=jnp.float32) / hd))
    freqs = pos[:, None] * inv_freq[None, :]
    emb = jnp.concatenate((freqs, freqs), axis=-1)
    cos = jnp.cos(emb)
    sin = jnp.sin(emb)

    bq2 = bq.reshape(1, -1)
    bk2 = bk.reshape(1, -1)
    bv2 = bv.reshape(1, -1)
    br2 = br.reshape(1, -1)
    bo2 = bo.reshape(1, -1)

    f32 = jnp.float32
    q, k, v, r = pl.pallas_call(
        lambda *refs: _proj_body(nheads, hd, *refs),
        grid=(n_qb, n_nb),
        in_specs=[
            pl.BlockSpec((BQ, hidden), lambda i, j: (i, 0)),
            pl.BlockSpec((hidden, BN), lambda i, j: (0, j)),
            pl.BlockSpec((hidden, BN), lambda i, j: (0, j)),
            pl.BlockSpec((hidden, BN), lambda i, j: (0, j)),
            pl.BlockSpec((hidden, nheads), lambda i, j: (0, 0)),
            pl.BlockSpec((1, BN), lambda i, j: (0, j)),
            pl.BlockSpec((1, BN), lambda i, j: (0, j)),
            pl.BlockSpec((1, BN), lambda i, j: (0, j)),
            pl.BlockSpec((1, nheads), lambda i, j: (0, 0)),
            pl.BlockSpec((BQ, hd), lambda i, j: (i, 0)),
            pl.BlockSpec((BQ, hd), lambda i, j: (i, 0)),
        ],
        out_specs=[
            pl.BlockSpec((BQ, BN), lambda i, j: (i, j)),
            pl.BlockSpec((BQ, BN), lambda i, j: (i, j)),
            pl.BlockSpec((BQ, BN), lambda i, j: (i, j)),
            pl.BlockSpec((BQ, nheads), lambda i, j: (i, 0)),
        ],
        out_shape=[
            jax.ShapeDtypeStruct((q_len, nheads * hd), f32),
            jax.ShapeDtypeStruct((q_len, nheads * hd), f32),
            jax.ShapeDtypeStruct((q_len, nheads * hd), f32),
            jax.ShapeDtypeStruct((q_len, nheads), f32),
        ],
        compiler_params=pltpu.CompilerParams(
            dimension_semantics=("parallel", "arbitrary")),
    )(x, Wq, Wk, Wv, Wr, bq2, bk2, bv2, br2, cos, sin)

    attn_out = pl.pallas_call(
        lambda *refs: _attn_body(keep, KT, *refs),
        grid=(nheads, n_qb),
        in_specs=[
            pl.BlockSpec((BQ, hd), lambda h, i: (i, h)),
            pl.BlockSpec((q_len, hd), lambda h, i: (0, h)),
            pl.BlockSpec((q_len, hd), lambda h, i: (0, h)),
            pl.BlockSpec((BQ, nheads), lambda h, i: (i, 0)),
        ],
        out_specs=pl.BlockSpec((BQ, hd), lambda h, i: (i, h)),
        out_shape=jax.ShapeDtypeStruct((q_len, nheads * hd), f32),
        scratch_shapes=[
            pltpu.VMEM((BQ, q_len), f32),
        ],
        compiler_params=pltpu.CompilerParams(
            dimension_semantics=("parallel", "parallel")),
    )(q, k, v, r)

    out = pl.pallas_call(
        _outproj_body,
        grid=(n_qb, n_nb),
        in_specs=[
            pl.BlockSpec((BQ, nheads * hd), lambda i, j: (i, 0)),
            pl.BlockSpec((nheads * hd, BN), lambda i, j: (0, j)),
            pl.BlockSpec((1, BN), lambda i, j: (0, j)),
        ],
        out_specs=pl.BlockSpec((BQ, BN), lambda i, j: (i, j)),
        out_shape=jax.ShapeDtypeStruct((q_len, hidden), f32),
        compiler_params=pltpu.CompilerParams(
            dimension_semantics=("parallel", "parallel")),
    )(attn_out, Wo, bo2)

    return out.reshape(bsz, q_len, hidden)


# speedup vs baseline: 1.4304x; 1.4304x over previous
import math
import statistics

import jax
import jax.numpy as jnp
from jax import lax
from jax.experimental import pallas as pl
from jax.experimental.pallas import tpu as pltpu

SPARSITY_RATIO = 0.5
VISION_SPARSITY_RATIO = 0.4


def _to_key(bits):
    return jnp.where(bits >= 0, bits, bits ^ jnp.int32(0x7FFFFFFF))


def _proj_body(nheads, head_dim, hs, wq, wk, wv, wr, bq, bk, bv, br, cos, sin,
               q_o, k_o, v_o, r_o):
    j = pl.program_id(0)
    x = hs[...]
    qt = jnp.dot(x, wq[...], preferred_element_type=jnp.float32) + bq[...]
    kt = jnp.dot(x, wk[...], preferred_element_type=jnp.float32) + bk[...]
    cosb = cos[...]
    sinb = sin[...]
    hd = head_dim
    half = hd // 2
    for hh in range(qt.shape[1] // hd):
        sl = slice(hh * hd, (hh + 1) * hd)
        qh = qt[:, sl]
        qr = jnp.concatenate([-qh[:, half:], qh[:, :half]], axis=1)
        q_o[:, sl] = qh * cosb + qr * sinb
        kh = kt[:, sl]
        kr = jnp.concatenate([-kh[:, half:], kh[:, :half]], axis=1)
        k_o[:, sl] = kh * cosb + kr * sinb
    v_o[...] = jnp.dot(x, wv[...], preferred_element_type=jnp.float32) + bv[...]

    @pl.when(j == 0)
    def _():
        r_o[...] = jax.nn.sigmoid(
            jnp.dot(x, wr[...], preferred_element_type=jnp.float32) + br[...])


def _attn_body(keep, kt_tile, q_r, k_r, v_r, r_r, o_r, m_scr):
    h = pl.program_id(0)
    bq = q_r.shape[0]
    qlen = k_r.shape[0]
    nkt = qlen // kt_tile
    hd = q_r.shape[1]
    scale = jnp.float32(1.0 / math.sqrt(hd))

    qv = q_r[...]
    rall = r_r[...]
    lane = lax.broadcasted_iota(jnp.int32, rall.shape, 1)
    rcol = jnp.sum(jnp.where(lane == h, rall, 0.0), axis=1, keepdims=True)
    rinv = 1.0 / rcol

    mmax = None
    mmin = None
    msum = None
    msum2 = None
    for t in range(nkt):
        ksl = pl.ds(t * kt_tile, kt_tile)
        kv = k_r[ksl, :]
        st = lax.dot_general(qv, kv, (((1,), (1,)), ((), ())),
                             preferred_element_type=jnp.float32) * scale
        mt = st * rcol
        m_scr[:, ksl] = mt
        tmax = jnp.max(mt, axis=1, keepdims=True)
        tmin = jnp.min(mt, axis=1, keepdims=True)
        tsum = jnp.sum(mt, axis=1, keepdims=True)
        tsum2 = jnp.sum(mt * mt, axis=1, keepdims=True)
        mmax = tmax if mmax is None else jnp.maximum(mmax, tmax)
        mmin = tmin if mmin is None else jnp.minimum(mmin, tmin)
        msum = tsum if msum is None else msum + tsum
        msum2 = tsum2 if msum2 is None else msum2 + tsum2

    kf = jnp.float32(keep)
    lo = _to_key(lax.bitcast_convert_type(mmin, jnp.int32))
    hi = _to_key(lax.bitcast_convert_type(mmax, jnp.int32))
    clo = jnp.full((bq, 1), jnp.float32(qlen))
    chi = jnp.zeros((bq, 1), jnp.float32)

    def count_ge(vmid):
        cnt = jnp.zeros((bq, 1), jnp.float32)
        for t in range(nkt):
            mt = m_scr[:, pl.ds(t * kt_tile, kt_tile)]
            cnt = cnt + jnp.sum(
                jnp.where(mt >= vmid, 1.0, 0.0), axis=1, keepdims=True)
        return cnt

    def update(st_, mid):
        lo_, hi_, clo_, chi_ = st_
        mid = jnp.clip(mid, lo_ + 1, hi_)
        vmidc = lax.bitcast_convert_type(_to_key(mid), jnp.float32)
        cnt = count_ge(vmidc)
        exact = cnt == kf
        ge = cnt >= kf
        lo2 = jnp.where(ge, mid, lo_)
        hi2 = jnp.where(exact, mid, jnp.where(ge, hi_, mid - 1))
        clo2 = jnp.where(ge, cnt, clo_)
        chi2 = jnp.where(ge, chi_, cnt)
        return (lo2, hi2, clo2, chi2)

    def interp_mid(st_):
        lo_, hi_, clo_, chi_ = st_
        vlo = lax.bitcast_convert_type(_to_key(lo_), jnp.float32)
        vhi = lax.bitcast_convert_type(_to_key(hi_), jnp.float32)
        frac = (clo_ - kf) / jnp.maximum(clo_ - chi_, 1.0)
        return _to_key(lax.bitcast_convert_type(vlo + frac * (vhi - vlo),
                                                jnp.int32))

    def bisect_mid(st_):
        lo_, hi_, _, _ = st_
        return (lo_ >> 1) + (hi_ >> 1) + ((lo_ | hi_) & 1)

    zq = jnp.float32(statistics.NormalDist().inv_cdf(
        max(1e-6, min(1.0 - 1e-6, (qlen - keep) / qlen))))
    mu = msum * jnp.float32(1.0 / qlen)
    var = jnp.maximum(msum2 * jnp.float32(1.0 / qlen) - mu * mu, 0.0)
    t0 = mu + jnp.sqrt(var) * zq
    state = (lo, hi, clo, chi)
    state = update(state, _to_key(lax.bitcast_convert_type(t0, jnp.int32)))
    for n in range(1, 8):
        state = update(state, bisect_mid(state) if n % 3 == 0
                       else interp_mid(state))

    def cond(st_):
        lo_, hi_, _, _ = st_
        return jnp.any(lo_ < hi_)

    def wbody(st_):
        st_ = update(st_, interp_mid(st_))
        return update(st_, bisect_mid(st_))

    lo, hi, _, _ = lax.while_loop(cond, wbody, state)
    vT = lax.bitcast_convert_type(_to_key(lo), jnp.float32)

    smax = mmax * rinv
    denom = jnp.zeros((bq, 1), jnp.float32)
    acc = jnp.zeros((bq, hd), jnp.float32)
    for t in range(nkt):
        ksl = pl.ds(t * kt_tile, kt_tile)
        mt = m_scr[:, ksl]
        st = mt * rinv
        p = jnp.where(mt >= vT, jnp.exp(st - smax), 0.0)
        denom = denom + jnp.sum(p, axis=1, keepdims=True)
        acc = acc + jnp.dot(p, v_r[ksl, :], preferred_element_type=jnp.float32)
    o_r[...] = acc / denom


def _outproj_body(x, wo, bo, o):
    o[...] = jnp.dot(x[...], wo[...], preferred_element_type=jnp.float32) + bo[...]


def kernel(hidden_states, Wq, bq, Wk, bk, Wv, bv, Wo, bo, Wr, br):
    bsz, q_len, hidden = hidden_states.shape
    nheads = Wr.shape[1]
    hd = Wq.shape[1] // nheads
    ratio = VISION_SPARSITY_RATIO if q_len > 512 else SPARSITY_RATIO
    keep = max(1, int(q_len * (1.0 - ratio)))

    BQ = min(512, q_len)
    BN = min(256, nheads * hd)
    KT = min(512, q_len)
    n_qb = q_len // BQ
    n_nb = (nheads * hd) // BN

    x = hidden_states.reshape(q_len, hidden)

    pos = jnp.arange(q_len, dtype=jnp.float32)
    inv_freq = 1.0 / (10000.0 ** (jnp.arange(0, hd, 2, dtype=jnp.float32) / hd))
    freqs = pos[:, None] * inv_freq[None, :]
    emb = jnp.concatenate((freqs, freqs), axis=-1)
    cos = jnp.cos(emb)
    sin = jnp.sin(emb)

    bq2 = bq.reshape(1, -1)
    bk2 = bk.reshape(1, -1)
    bv2 = bv.reshape(1, -1)
    br2 = br.reshape(1, -1)
    bo2 = bo.reshape(1, -1)

    f32 = jnp.float32
    BNP = min(128, nheads * hd)
    n_nbp = (nheads * hd) // BNP
    q, k, v, r = pl.pallas_call(
        lambda *refs: _proj_body(nheads, hd, *refs),
        grid=(n_nbp,),
        in_specs=[
            pl.BlockSpec((q_len, hidden), lambda j: (0, 0)),
            pl.BlockSpec((hidden, BNP), lambda j: (0, j)),
            pl.BlockSpec((hidden, BNP), lambda j: (0, j)),
            pl.BlockSpec((hidden, BNP), lambda j: (0, j)),
            pl.BlockSpec((hidden, nheads), lambda j: (0, 0)),
            pl.BlockSpec((1, BNP), lambda j: (0, j)),
            pl.BlockSpec((1, BNP), lambda j: (0, j)),
            pl.BlockSpec((1, BNP), lambda j: (0, j)),
            pl.BlockSpec((1, nheads), lambda j: (0, 0)),
            pl.BlockSpec((q_len, hd), lambda j: (0, 0)),
            pl.BlockSpec((q_len, hd), lambda j: (0, 0)),
        ],
        out_specs=[
            pl.BlockSpec((q_len, BNP), lambda j: (0, j)),
            pl.BlockSpec((q_len, BNP), lambda j: (0, j)),
            pl.BlockSpec((q_len, BNP), lambda j: (0, j)),
            pl.BlockSpec((q_len, nheads), lambda j: (0, 0)),
        ],
        out_shape=[
            jax.ShapeDtypeStruct((q_len, nheads * hd), f32),
            jax.ShapeDtypeStruct((q_len, nheads * hd), f32),
            jax.ShapeDtypeStruct((q_len, nheads * hd), f32),
            jax.ShapeDtypeStruct((q_len, nheads), f32),
        ],
        compiler_params=pltpu.CompilerParams(
            dimension_semantics=("arbitrary",)),
    )(x, Wq, Wk, Wv, Wr, bq2, bk2, bv2, br2, cos, sin)

    attn_out = pl.pallas_call(
        lambda *refs: _attn_body(keep, KT, *refs),
        grid=(nheads, n_qb),
        in_specs=[
            pl.BlockSpec((BQ, hd), lambda h, i: (i, h)),
            pl.BlockSpec((q_len, hd), lambda h, i: (0, h)),
            pl.BlockSpec((q_len, hd), lambda h, i: (0, h)),
            pl.BlockSpec((BQ, nheads), lambda h, i: (i, 0)),
        ],
        out_specs=pl.BlockSpec((BQ, hd), lambda h, i: (i, h)),
        out_shape=jax.ShapeDtypeStruct((q_len, nheads * hd), f32),
        scratch_shapes=[
            pltpu.VMEM((BQ, q_len), f32),
        ],
        compiler_params=pltpu.CompilerParams(
            dimension_semantics=("parallel", "parallel")),
    )(q, k, v, r)

    out = pl.pallas_call(
        _outproj_body,
        grid=(n_nb,),
        in_specs=[
            pl.BlockSpec((q_len, nheads * hd), lambda j: (0, 0)),
            pl.BlockSpec((nheads * hd, BN), lambda j: (0, j)),
            pl.BlockSpec((1, BN), lambda j: (0, j)),
        ],
        out_specs=pl.BlockSpec((q_len, BN), lambda j: (0, j)),
        out_shape=jax.ShapeDtypeStruct((q_len, hidden), f32),
        compiler_params=pltpu.CompilerParams(
            dimension_semantics=("parallel",)),
    )(attn_out, Wo, bo2)

    return out.reshape(bsz, q_len, hidden)


# speedup vs baseline: 1.4370x; 1.0046x over previous
import math
import statistics

import jax
import jax.numpy as jnp
from jax import lax
from jax.experimental import pallas as pl
from jax.experimental.pallas import tpu as pltpu

SPARSITY_RATIO = 0.5
VISION_SPARSITY_RATIO = 0.4


def _to_key(bits):
    return jnp.where(bits >= 0, bits, bits ^ jnp.int32(0x7FFFFFFF))


def _proj_body(nheads, head_dim, hs, wq, wk, wv, wr, bq, bk, bv, br, cos, sin,
               q_o, k_o, v_o, r_o):
    j = pl.program_id(0)
    x = hs[...]
    qt = jnp.dot(x, wq[...], preferred_element_type=jnp.float32) + bq[...]
    kt = jnp.dot(x, wk[...], preferred_element_type=jnp.float32) + bk[...]
    cosb = cos[...]
    sinb = sin[...]
    hd = head_dim
    half = hd // 2
    for hh in range(qt.shape[1] // hd):
        sl = slice(hh * hd, (hh + 1) * hd)
        qh = qt[:, sl]
        qr = jnp.concatenate([-qh[:, half:], qh[:, :half]], axis=1)
        q_o[:, sl] = qh * cosb + qr * sinb
        kh = kt[:, sl]
        kr = jnp.concatenate([-kh[:, half:], kh[:, :half]], axis=1)
        k_o[:, sl] = kh * cosb + kr * sinb
    v_o[...] = jnp.dot(x, wv[...], preferred_element_type=jnp.float32) + bv[...]

    @pl.when(j == 0)
    def _():
        r_o[...] = jax.nn.sigmoid(
            jnp.dot(x, wr[...], preferred_element_type=jnp.float32) + br[...])


def _attn_body(keep, kt_tile, q_r, k_r, v_r, r_r, o_r, m_scr):
    h = pl.program_id(0)
    bq = q_r.shape[0]
    qlen = k_r.shape[0]
    nkt = qlen // kt_tile
    hd = q_r.shape[1]
    scale = jnp.float32(1.0 / math.sqrt(hd))

    qv = q_r[...]
    rall = r_r[...]
    lane = lax.broadcasted_iota(jnp.int32, rall.shape, 1)
    rcol = jnp.sum(jnp.where(lane == h, rall, 0.0), axis=1, keepdims=True)
    rinv = 1.0 / rcol

    mmax = None
    mmin = None
    msum = None
    msum2 = None
    for t in range(nkt):
        ksl = pl.ds(t * kt_tile, kt_tile)
        kv = k_r[ksl, :]
        st = lax.dot_general(qv, kv, (((1,), (1,)), ((), ())),
                             preferred_element_type=jnp.float32) * scale
        mt = st * rcol
        m_scr[:, ksl] = mt
        tmax = jnp.max(mt, axis=1, keepdims=True)
        tmin = jnp.min(mt, axis=1, keepdims=True)
        tsum = jnp.sum(mt, axis=1, keepdims=True)
        tsum2 = jnp.sum(mt * mt, axis=1, keepdims=True)
        mmax = tmax if mmax is None else jnp.maximum(mmax, tmax)
        mmin = tmin if mmin is None else jnp.minimum(mmin, tmin)
        msum = tsum if msum is None else msum + tsum
        msum2 = tsum2 if msum2 is None else msum2 + tsum2

    kf = jnp.float32(keep)
    lo = _to_key(lax.bitcast_convert_type(mmin, jnp.int32))
    hi = _to_key(lax.bitcast_convert_type(mmax, jnp.int32))
    clo = jnp.full((bq, 1), jnp.float32(qlen))
    chi = jnp.zeros((bq, 1), jnp.float32)

    def count_ge(vmid):
        cnt = jnp.zeros((bq, 1), jnp.float32)
        for t in range(nkt):
            mt = m_scr[:, pl.ds(t * kt_tile, kt_tile)]
            cnt = cnt + jnp.sum(
                jnp.where(mt >= vmid, 1.0, 0.0), axis=1, keepdims=True)
        return cnt

    def update(st_, mid):
        lo_, hi_, clo_, chi_ = st_
        mid = jnp.clip(mid, lo_ + 1, hi_)
        vmidc = lax.bitcast_convert_type(_to_key(mid), jnp.float32)
        cnt = count_ge(vmidc)
        exact = cnt == kf
        ge = cnt >= kf
        lo2 = jnp.where(ge, mid, lo_)
        hi2 = jnp.where(exact, mid, jnp.where(ge, hi_, mid - 1))
        clo2 = jnp.where(ge, cnt, clo_)
        chi2 = jnp.where(ge, chi_, cnt)
        return (lo2, hi2, clo2, chi2)

    def interp_mid(st_):
        lo_, hi_, clo_, chi_ = st_
        vlo = lax.bitcast_convert_type(_to_key(lo_), jnp.float32)
        vhi = lax.bitcast_convert_type(_to_key(hi_), jnp.float32)
        frac = (clo_ - kf) / jnp.maximum(clo_ - chi_, 1.0)
        return _to_key(lax.bitcast_convert_type(vlo + frac * (vhi - vlo),
                                                jnp.int32))

    def bisect_mid(st_):
        lo_, hi_, _, _ = st_
        return (lo_ >> 1) + (hi_ >> 1) + ((lo_ | hi_) & 1)

    zq = jnp.float32(statistics.NormalDist().inv_cdf(
        max(1e-6, min(1.0 - 1e-6, (qlen - keep) / qlen))))
    mu = msum * jnp.float32(1.0 / qlen)
    var = jnp.maximum(msum2 * jnp.float32(1.0 / qlen) - mu * mu, 0.0)
    t0 = mu + jnp.sqrt(var) * zq
    state = (lo, hi, clo, chi)
    state = update(state, _to_key(lax.bitcast_convert_type(t0, jnp.int32)))
    for n in range(1, 8):
        state = update(state, bisect_mid(state) if n % 3 == 0
                       else interp_mid(state))

    def cond(st_):
        lo_, hi_, _, _ = st_
        return jnp.any(lo_ < hi_)

    def wbody(st_):
        st_ = update(st_, interp_mid(st_))
        return update(st_, bisect_mid(st_))

    lo, hi, _, _ = lax.while_loop(cond, wbody, state)
    vT = lax.bitcast_convert_type(_to_key(lo), jnp.float32)

    smax = mmax * rinv
    denom = jnp.zeros((bq, 1), jnp.float32)
    acc = jnp.zeros((bq, hd), jnp.float32)
    for t in range(nkt):
        ksl = pl.ds(t * kt_tile, kt_tile)
        mt = m_scr[:, ksl]
        st = mt * rinv
        p = jnp.where(mt >= vT, jnp.exp(st - smax), 0.0)
        denom = denom + jnp.sum(p, axis=1, keepdims=True)
        acc = acc + jnp.dot(p, v_r[ksl, :], preferred_element_type=jnp.float32)
    o_r[...] = acc / denom


def _outproj_body(x, wo, bo, o):
    o[...] = jnp.dot(x[...], wo[...], preferred_element_type=jnp.float32) + bo[...]


def kernel(hidden_states, Wq, bq, Wk, bk, Wv, bv, Wo, bo, Wr, br):
    bsz, q_len, hidden = hidden_states.shape
    nheads = Wr.shape[1]
    hd = Wq.shape[1] // nheads
    ratio = VISION_SPARSITY_RATIO if q_len > 512 else SPARSITY_RATIO
    keep = max(1, int(q_len * (1.0 - ratio)))

    BQ = min(512, q_len)
    BN = min(256, nheads * hd)
    KT = min(1024, q_len)
    n_qb = q_len // BQ
    n_nb = (nheads * hd) // BN

    x = hidden_states.reshape(q_len, hidden)

    pos = jnp.arange(q_len, dtype=jnp.float32)
    inv_freq = 1.0 / (10000.0 ** (jnp.arange(0, hd, 2, dtype=jnp.float32) / hd))
    freqs = pos[:, None] * inv_freq[None, :]
    emb = jnp.concatenate((freqs, freqs), axis=-1)
    cos = jnp.cos(emb)
    sin = jnp.sin(emb)

    bq2 = bq.reshape(1, -1)
    bk2 = bk.reshape(1, -1)
    bv2 = bv.reshape(1, -1)
    br2 = br.reshape(1, -1)
    bo2 = bo.reshape(1, -1)

    f32 = jnp.float32
    BNP = min(128, nheads * hd)
    n_nbp = (nheads * hd) // BNP
    q, k, v, r = pl.pallas_call(
        lambda *refs: _proj_body(nheads, hd, *refs),
        grid=(n_nbp,),
        in_specs=[
            pl.BlockSpec((q_len, hidden), lambda j: (0, 0)),
            pl.BlockSpec((hidden, BNP), lambda j: (0, j)),
            pl.BlockSpec((hidden, BNP), lambda j: (0, j)),
            pl.BlockSpec((hidden, BNP), lambda j: (0, j)),
            pl.BlockSpec((hidden, nheads), lambda j: (0, 0)),
            pl.BlockSpec((1, BNP), lambda j: (0, j)),
            pl.BlockSpec((1, BNP), lambda j: (0, j)),
            pl.BlockSpec((1, BNP), lambda j: (0, j)),
            pl.BlockSpec((1, nheads), lambda j: (0, 0)),
            pl.BlockSpec((q_len, hd), lambda j: (0, 0)),
            pl.BlockSpec((q_len, hd), lambda j: (0, 0)),
        ],
        out_specs=[
            pl.BlockSpec((q_len, BNP), lambda j: (0, j)),
            pl.BlockSpec((q_len, BNP), lambda j: (0, j)),
            pl.BlockSpec((q_len, BNP), lambda j: (0, j)),
            pl.BlockSpec((q_len, nheads), lambda j: (0, 0)),
        ],
        out_shape=[
            jax.ShapeDtypeStruct((q_len, nheads * hd), f32),
            jax.ShapeDtypeStruct((q_len, nheads * hd), f32),
            jax.ShapeDtypeStruct((q_len, nheads * hd), f32),
            jax.ShapeDtypeStruct((q_len, nheads), f32),
        ],
        compiler_params=pltpu.CompilerParams(
            dimension_semantics=("arbitrary",)),
    )(x, Wq, Wk, Wv, Wr, bq2, bk2, bv2, br2, cos, sin)

    attn_out = pl.pallas_call(
        lambda *refs: _attn_body(keep, KT, *refs),
        grid=(nheads, n_qb),
        in_specs=[
            pl.BlockSpec((BQ, hd), lambda h, i: (i, h)),
            pl.BlockSpec((q_len, hd), lambda h, i: (0, h)),
            pl.BlockSpec((q_len, hd), lambda h, i: (0, h)),
            pl.BlockSpec((BQ, nheads), lambda h, i: (i, 0)),
        ],
        out_specs=pl.BlockSpec((BQ, hd), lambda h, i: (i, h)),
        out_shape=jax.ShapeDtypeStruct((q_len, nheads * hd), f32),
        scratch_shapes=[
            pltpu.VMEM((BQ, q_len), f32),
        ],
        compiler_params=pltpu.CompilerParams(
            dimension_semantics=("parallel", "parallel")),
    )(q, k, v, r)

    out = pl.pallas_call(
        _outproj_body,
        grid=(n_nb,),
        in_specs=[
            pl.BlockSpec((q_len, nheads * hd), lambda j: (0, 0)),
            pl.BlockSpec((nheads * hd, BN), lambda j: (0, j)),
            pl.BlockSpec((1, BN), lambda j: (0, j)),
        ],
        out_specs=pl.BlockSpec((q_len, BN), lambda j: (0, j)),
        out_shape=jax.ShapeDtypeStruct((q_len, hidden), f32),
        compiler_params=pltpu.CompilerParams(
            dimension_semantics=("parallel",)),
    )(attn_out, Wo, bo2)

    return out.reshape(bsz, q_len, hidden)
